# baseline (speedup 1.0000x reference)
import functools

import jax
import jax.numpy as jnp
from jax import lax
from jax.experimental import pallas as pl
from jax.experimental.pallas import tpu as pltpu
from jax.experimental.pallas import tpu_sc as plsc

NUM_USERS = 1000000
NUM_ITEMS = 1000000
EMBED_DIM = 64
BATCH = 16384

_NC = 2
_NS = 16
_NW = _NC * _NS
_BPW = BATCH // _NW
_HALF = _BPW // 2
_CHUNK = 128
_GROUP = 16
_L = 16
_NV = EMBED_DIM // _L


def _sc_body(uj_hbm, pj_hbm, nj_hbm, uraw_hbm, praw_hbm, nraw_hbm,
             utab2_hbm, itab2_hbm,
             pos_out, neg_out, reg_out,
             ujidx_v, pjidx_v, njidx_v, uraw_v, praw_v, nraw_v,
             urows_v, prows_v, nrows_v,
             csp_v, csn_v, psc_v, nsc_v, acc_v, sem):
    wid = lax.axis_index("s") * _NC + lax.axis_index("c")

    pltpu.sync_copy(uj_hbm.at[wid], ujidx_v)
    pltpu.sync_copy(pj_hbm.at[wid], pjidx_v)
    pltpu.sync_copy(nj_hbm.at[wid], njidx_v)
    pltpu.sync_copy(uraw_hbm.at[wid], uraw_v)
    pltpu.sync_copy(praw_hbm.at[wid], praw_v)
    pltpu.sync_copy(nraw_hbm.at[wid], nraw_v)

    def fire(q):
        sl = pl.ds((q % 2) * _CHUNK, _CHUNK)
        pltpu.make_async_copy(
            utab2_hbm.at[ujidx_v.at[q]], urows_v.at[sl], sem).start()
        pltpu.make_async_copy(
            itab2_hbm.at[pjidx_v.at[q]], prows_v.at[sl], sem).start()
        pltpu.make_async_copy(
            itab2_hbm.at[njidx_v.at[q]], nrows_v.at[sl], sem).start()

    def drain(q):
        sl = pl.ds((q % 2) * _CHUNK, _CHUNK)
        pltpu.make_async_copy(
            utab2_hbm.at[ujidx_v.at[0]], urows_v.at[sl], sem).wait()
        pltpu.make_async_copy(
            itab2_hbm.at[pjidx_v.at[0]], prows_v.at[sl], sem).wait()
        pltpu.make_async_copy(
            itab2_hbm.at[njidx_v.at[0]], nrows_v.at[sl], sem).wait()

    lane15 = lax.iota(jnp.int32, _L) * _L + (_L - 1)

    def compute_group(q, g, acc):
        iu = uraw_v[pl.ds(q * _CHUNK + g * _GROUP, _GROUP)]
        ip = praw_v[pl.ds(q * _CHUNK + g * _GROUP, _GROUP)]
        iq = nraw_v[pl.ds(q * _CHUNK + g * _GROUP, _GROUP)]
        for i in range(_GROUP):
            b = (q % 2) * _CHUNK + g * _GROUP + i
            ou = (iu[i] & 1) * EMBED_DIM
            op = (ip[i] & 1) * EMBED_DIM
            on = (iq[i] & 1) * EMBED_DIM
            dot_p = None
            dot_n = None
            for c in range(_NV):
                u = urows_v[b, pl.ds(ou + c * _L, _L)]
                p = prows_v[b, pl.ds(op + c * _L, _L)]
                n = nrows_v[b, pl.ds(on + c * _L, _L)]
                acc = acc + u * u + p * p + n * n
                if dot_p is None:
                    dot_p = u * p
                    dot_n = u * n
                else:
                    dot_p = dot_p + u * p
                    dot_n = dot_n + u * n
            csp_v[pl.ds(i * _L, _L)] = plsc.cumsum(dot_p)
            csn_v[pl.ds(i * _L, _L)] = plsc.cumsum(dot_n)
        base = q * _CHUNK + g * _GROUP
        psc_v[pl.ds(base, _GROUP)] = plsc.load_gather(csp_v, [lane15])
        nsc_v[pl.ds(base, _GROUP)] = plsc.load_gather(csn_v, [lane15])
        return acc

    acc = jnp.zeros((_L,), jnp.float32)
    fire(0)
    for q in range(_BPW // _CHUNK):
        if q < _BPW // _CHUNK - 1:
            fire(q + 1)
        drain(q)
        acc = lax.fori_loop(
            0, _CHUNK // _GROUP,
            functools.partial(compute_group, q), acc, unroll=False)
    acc_v[...] = acc

    base = wid * _BPW
    pltpu.sync_copy(psc_v, pos_out.at[pl.ds(base, _BPW)])
    pltpu.sync_copy(nsc_v, neg_out.at[pl.ds(base, _BPW)])
    pltpu.sync_copy(acc_v, reg_out.at[wid])


@jax.jit
def _bprmf_sc(uj, pj, nj, uraw, praw, nraw, user_table, item_table):
    utab2 = user_table.reshape(NUM_USERS // 2, 2 * EMBED_DIM)
    itab2 = item_table.reshape(NUM_ITEMS // 2, 2 * EMBED_DIM)
    mesh = plsc.VectorSubcoreMesh(core_axis_name="c", subcore_axis_name="s")
    f = functools.partial(
        pl.kernel,
        mesh=mesh,
        compiler_params=pltpu.CompilerParams(
            needs_layout_passes=False, use_tc_tiling_on_sc=False),
        out_type=(
            jax.ShapeDtypeStruct((BATCH,), jnp.float32),
            jax.ShapeDtypeStruct((BATCH,), jnp.float32),
            jax.ShapeDtypeStruct((_NW, _L), jnp.float32),
        ),
        scratch_types=[
            pltpu.VMEM((4, _CHUNK), jnp.int32),
            pltpu.VMEM((4, _CHUNK), jnp.int32),
            pltpu.VMEM((4, _CHUNK), jnp.int32),
            pltpu.VMEM((_BPW,), jnp.int32),
            pltpu.VMEM((_BPW,), jnp.int32),
            pltpu.VMEM((_BPW,), jnp.int32),
            pltpu.VMEM((2 * _CHUNK, 2 * EMBED_DIM), jnp.float32),
            pltpu.VMEM((2 * _CHUNK, 2 * EMBED_DIM), jnp.float32),
            pltpu.VMEM((2 * _CHUNK, 2 * EMBED_DIM), jnp.float32),
            pltpu.VMEM((_GROUP * _L,), jnp.float32),
            pltpu.VMEM((_GROUP * _L,), jnp.float32),
            pltpu.VMEM((_BPW,), jnp.float32),
            pltpu.VMEM((_BPW,), jnp.float32),
            pltpu.VMEM((_L,), jnp.float32),
            pltpu.SemaphoreType.DMA,
        ],
    )(_sc_body)
    return f(uj, pj, nj, uraw, praw, nraw, utab2, itab2)


def kernel(users, pos_items, neg_items, user_table, item_table):
    uj = (users >> 1).reshape(_NW, 4, _CHUNK)
    pj = (pos_items >> 1).reshape(_NW, 4, _CHUNK)
    nj = (neg_items >> 1).reshape(_NW, 4, _CHUNK)
    uraw = users.reshape(_NW, _BPW)
    praw = pos_items.reshape(_NW, _BPW)
    nraw = neg_items.reshape(_NW, _BPW)
    pos_scores, neg_scores, reg_part = _bprmf_sc(
        uj, pj, nj, uraw, praw, nraw, user_table, item_table)
    reg_loss = 0.5 * jnp.sum(reg_part) / float(BATCH)
    return (pos_scores, neg_scores, reg_loss)


# speedup vs baseline: 2.0690x; 2.0690x over previous
import functools

import jax
import jax.numpy as jnp
from jax import lax
from jax.experimental import pallas as pl
from jax.experimental.pallas import tpu as pltpu
from jax.experimental.pallas import tpu_sc as plsc

NUM_ROWS = 1000000
EMBED_DIM = 64
BATCH = 16384

_NC = 2
_NS = 16
_NW = _NC * _NS
_L = 16
_NV = EMBED_DIM // _L
_BW = 128
_NB = NUM_ROWS // _BW
_RAG = NUM_ROWS - _NB * _BW
_ICAP = 2048
_BPW = BATCH // _NW
_GROUP = 16


def _scan_pass(idx_hbm, idx_v, reqi_v, reqb_v, wid, b_off, off0):
    pltpu.sync_copy(idx_hbm, idx_v)
    iota = lax.iota(jnp.int32, _L)

    def body(k, off):
        iv = idx_v[pl.ds(k * _L, _L)]
        blkv = lax.shift_right_logical(iv, 7)
        m = (blkv & (_NW - 1)) == wid
        bv = k * _L + iota + b_off
        plsc.store_compressed(reqi_v.at[pl.ds(off, _L)], iv, mask=m)
        plsc.store_compressed(reqb_v.at[pl.ds(off, _L)], bv, mask=m)
        cnt = plsc.all_reduce_population_count(m)
        return off + cnt[0]

    return lax.fori_loop(0, BATCH // _L, body, off0, unroll=False)


def _extract_body(users_hbm, pos_hbm, neg_hbm, utabT_hbm, itabT_hbm,
                  uemb, pemb, nemb,
                  idx_v, reqi_v, reqb_v, hiti_v, hitb_v,
                  blk0_v, blk1_v, ragblk_v, stage_v, semb0, semb1, semr):
    wid = lax.axis_index("s") * _NC + lax.axis_index("c")

    def run_table_pass(tabT_hbm, nreq, item_pass):
        iota = lax.iota(jnp.int32, _L)
        nchunk = lax.div(nreq + _L - 1, jnp.int32(_L))
        def fire(blk_v, semb, j):
            it = wid + _NW * j
            @pl.when(it < _NB)
            def _():
                pltpu.make_async_copy(
                    tabT_hbm.at[:, pl.ds(it * _BW, _BW)], blk_v, semb).start()

        def waitblk(blk_v, semb, j):
            it = wid + _NW * j
            @pl.when(it < _NB)
            def _():
                pltpu.make_async_copy(
                    tabT_hbm.at[:, pl.ds(0, _BW)], blk_v, semb).wait()

        def emit_row(blk_v, r):
            iv = hiti_v[pl.ds(r, _L)]
            bv = hitb_v[pl.ds(r, _L)]
            il = iv[0] & (_BW - 1)
            bb = bv[0]
            ilv = il + jnp.zeros((_L,), jnp.int32)
            slot = r * EMBED_DIM
            for c in range(_NV):
                dv = lax.iota(jnp.int32, _L) + c * _L
                vals = plsc.load_gather(blk_v, [dv, ilv])
                stage_v[pl.ds(slot + c * _L, _L)] = vals
            src = stage_v.at[pl.ds(slot, EMBED_DIM)]
            if item_pass:
                @pl.when(bb < BATCH)
                def _():
                    pltpu.make_async_copy(
                        src, pemb.at[pl.ds(bb * EMBED_DIM, EMBED_DIM)],
                        semr).start()
                @pl.when(bb >= BATCH)
                def _():
                    pltpu.make_async_copy(
                        src,
                        nemb.at[pl.ds((bb - BATCH) * EMBED_DIM, EMBED_DIM)],
                        semr).start()
            else:
                pltpu.make_async_copy(
                    src, uemb.at[pl.ds(bb * EMBED_DIM, EMBED_DIM)],
                    semr).start()

        def gather_hits(it):
            def body(t, nh):
                base = t * _L
                iv = reqi_v[pl.ds(base, _L)]
                bv = reqb_v[pl.ds(base, _L)]
                itv = lax.shift_right_logical(iv, 7)
                m = (itv == it) & ((base + iota) < nreq)
                plsc.store_compressed(hiti_v.at[pl.ds(nh, _L)], iv, mask=m)
                plsc.store_compressed(hitb_v.at[pl.ds(nh, _L)], bv, mask=m)
                cnt = plsc.all_reduce_population_count(m)
                return nh + cnt[0]
            return lax.fori_loop(0, nchunk, body, jnp.int32(0),
                                 unroll=False)

        def drain_rows(nh):
            def body(r, _):
                pltpu.make_async_copy(
                    stage_v.at[pl.ds(0, EMBED_DIM)],
                    uemb.at[pl.ds(0, EMBED_DIM)], semr).wait()
                return 0
            lax.fori_loop(0, nh, body, 0, unroll=False)

        def process(blk_v, j):
            it = wid + _NW * j
            @pl.when(it < _NB)
            def _():
                nh = gather_hits(it)
                def body(r, _):
                    emit_row(blk_v, r)
                    return 0
                lax.fori_loop(0, nh, body, 0, unroll=False)
                drain_rows(nh)

        fire(blk0_v, semb0, 0)
        fire(blk1_v, semb1, 1)

        def pair_body(jj, _):
            j0 = 2 * jj
            waitblk(blk0_v, semb0, j0)
            process(blk0_v, j0)
            fire(blk0_v, semb0, j0 + 2)
            waitblk(blk1_v, semb1, j0 + 1)
            process(blk1_v, j0 + 1)
            fire(blk1_v, semb1, j0 + 3)
            return 0

        lax.fori_loop(0, 123, pair_body, 0, unroll=False)

        @pl.when(wid == (_NB % _NW))
        def _():
            pltpu.sync_copy(
                tabT_hbm.at[:, pl.ds(_NB * _BW, _RAG)], ragblk_v)
            nh = gather_hits(jnp.int32(_NB))
            def body(r, _):
                emit_row(ragblk_v, r)
                return 0
            lax.fori_loop(0, nh, body, 0, unroll=False)
            drain_rows(nh)

    nu = _scan_pass(users_hbm, idx_v, reqi_v, reqb_v, wid, 0, jnp.int32(0))
    run_table_pass(utabT_hbm, nu, item_pass=False)

    np_ = _scan_pass(pos_hbm, idx_v, reqi_v, reqb_v, wid, 0, jnp.int32(0))
    ni = _scan_pass(neg_hbm, idx_v, reqi_v, reqb_v, wid, BATCH, np_)
    run_table_pass(itabT_hbm, ni, item_pass=True)


def _compute_body(uemb_hbm, pemb_hbm, nemb_hbm,
                  pos_out, neg_out, reg_out,
                  urows_v, prows_v, nrows_v,
                  csp_v, csn_v, psc_v, nsc_v, acc_v):
    wid = lax.axis_index("s") * _NC + lax.axis_index("c")
    base = wid * _BPW
    fbase = base * EMBED_DIM
    pltpu.sync_copy(uemb_hbm.at[pl.ds(fbase, _BPW * EMBED_DIM)], urows_v)
    pltpu.sync_copy(pemb_hbm.at[pl.ds(fbase, _BPW * EMBED_DIM)], prows_v)
    pltpu.sync_copy(nemb_hbm.at[pl.ds(fbase, _BPW * EMBED_DIM)], nrows_v)

    lane15 = lax.iota(jnp.int32, _L) * _L + (_L - 1)

    def group_body(g, acc):
        for i in range(_GROUP):
            b = g * _GROUP + i
            dot_p = None
            dot_n = None
            for c in range(_NV):
                sl = pl.ds(b * EMBED_DIM + c * _L, _L)
                u = urows_v[sl]
                p = prows_v[sl]
                n = nrows_v[sl]
                acc = acc + u * u + p * p + n * n
                if dot_p is None:
                    dot_p = u * p
                    dot_n = u * n
                else:
                    dot_p = dot_p + u * p
                    dot_n = dot_n + u * n
            csp_v[pl.ds(i * _L, _L)] = plsc.cumsum(dot_p)
            csn_v[pl.ds(i * _L, _L)] = plsc.cumsum(dot_n)
        psc_v[pl.ds(g * _GROUP, _GROUP)] = plsc.load_gather(csp_v, [lane15])
        nsc_v[pl.ds(g * _GROUP, _GROUP)] = plsc.load_gather(csn_v, [lane15])
        return acc

    acc = lax.fori_loop(0, _BPW // _GROUP, group_body,
                        jnp.zeros((_L,), jnp.float32), unroll=False)
    acc_v[...] = acc

    pltpu.sync_copy(psc_v, pos_out.at[pl.ds(base, _BPW)])
    pltpu.sync_copy(nsc_v, neg_out.at[pl.ds(base, _BPW)])
    pltpu.sync_copy(acc_v, reg_out.at[wid])


@jax.jit
def _bprmf_sc(users, pos_items, neg_items, user_table, item_table):
    utabT = user_table.T
    itabT = item_table.T
    mesh = plsc.VectorSubcoreMesh(core_axis_name="c", subcore_axis_name="s")
    params = pltpu.CompilerParams(
        needs_layout_passes=False, use_tc_tiling_on_sc=True)

    extract = functools.partial(
        pl.kernel,
        mesh=mesh,
        compiler_params=params,
        out_type=(
            jax.ShapeDtypeStruct((BATCH * EMBED_DIM,), jnp.float32),
            jax.ShapeDtypeStruct((BATCH * EMBED_DIM,), jnp.float32),
            jax.ShapeDtypeStruct((BATCH * EMBED_DIM,), jnp.float32),
        ),
        scratch_types=[
            pltpu.VMEM((BATCH,), jnp.int32),
            pltpu.VMEM((_ICAP + _L,), jnp.int32),
            pltpu.VMEM((_ICAP + _L,), jnp.int32),
            pltpu.VMEM((64 + _L,), jnp.int32),
            pltpu.VMEM((64 + _L,), jnp.int32),
            pltpu.VMEM((EMBED_DIM, _BW), jnp.float32),
            pltpu.VMEM((EMBED_DIM, _BW), jnp.float32),
            pltpu.VMEM((EMBED_DIM, _RAG), jnp.float32),
            pltpu.VMEM((64 * EMBED_DIM,), jnp.float32),
            pltpu.SemaphoreType.DMA,
            pltpu.SemaphoreType.DMA,
            pltpu.SemaphoreType.DMA,
        ],
    )(_extract_body)
    uemb, pemb, nemb = extract(users, pos_items, neg_items, utabT, itabT)

    compute = functools.partial(
        pl.kernel,
        mesh=mesh,
        compiler_params=params,
        out_type=(
            jax.ShapeDtypeStruct((BATCH,), jnp.float32),
            jax.ShapeDtypeStruct((BATCH,), jnp.float32),
            jax.ShapeDtypeStruct((_NW, _L), jnp.float32),
        ),
        scratch_types=[
            pltpu.VMEM((_BPW * EMBED_DIM,), jnp.float32),
            pltpu.VMEM((_BPW * EMBED_DIM,), jnp.float32),
            pltpu.VMEM((_BPW * EMBED_DIM,), jnp.float32),
            pltpu.VMEM((_GROUP * _L,), jnp.float32),
            pltpu.VMEM((_GROUP * _L,), jnp.float32),
            pltpu.VMEM((_BPW,), jnp.float32),
            pltpu.VMEM((_BPW,), jnp.float32),
            pltpu.VMEM((_L,), jnp.float32),
        ],
    )(_compute_body)
    return compute(uemb, pemb, nemb)


def kernel(users, pos_items, neg_items, user_table, item_table):
    pos_scores, neg_scores, reg_part = _bprmf_sc(
        users, pos_items, neg_items, user_table, item_table)
    reg_loss = 0.5 * jnp.sum(reg_part) / float(BATCH)
    return (pos_scores, neg_scores, reg_loss)


# speedup vs baseline: 2.5992x; 1.2562x over previous
import functools

import jax
import jax.numpy as jnp
from jax import lax
from jax.experimental import pallas as pl
from jax.experimental.pallas import tpu as pltpu
from jax.experimental.pallas import tpu_sc as plsc

NUM_ROWS = 1000000
EMBED_DIM = 64
BATCH = 16384

_NC = 2
_NS = 16
_NW = _NC * _NS
_L = 16
_NV = EMBED_DIM // _L
_BW = 128
_NB = NUM_ROWS // _BW
_RAG = NUM_ROWS - _NB * _BW
_ICAP = 2048
_BPW = BATCH // _NW
_GROUP = 16


def _scan_pass(idx_hbm, idx_v, reqi_v, reqb_v, wid, b_off, off0):
    pltpu.sync_copy(idx_hbm, idx_v)
    iota = lax.iota(jnp.int32, _L)

    def body(k, off):
        iv = idx_v[pl.ds(k * _L, _L)]
        blkv = lax.shift_right_logical(iv, 7)
        m = (blkv & (_NW - 1)) == wid
        bv = k * _L + iota + b_off
        plsc.store_compressed(reqi_v.at[pl.ds(off, _L)], iv, mask=m)
        plsc.store_compressed(reqb_v.at[pl.ds(off, _L)], bv, mask=m)
        cnt = plsc.all_reduce_population_count(m)
        return off + cnt[0]

    return lax.fori_loop(0, BATCH // _L, body, off0, unroll=False)


_GCAP = 192
_NG = 16
_SSLOT = 32


def _extract_body(users_hbm, pos_hbm, neg_hbm, utabT_hbm, itabT_hbm,
                  uemb, pemb, nemb,
                  idx_v, reqi_v, reqb_v, gi_v, gb_v, hiti_v, hitb_v,
                  blk0_v, blk1_v, ragblk_v, stage_v, semb0, semb1, semr):
    wid = lax.axis_index("s") * _NC + lax.axis_index("c")
    iota = lax.iota(jnp.int32, _L)

    def partition(nreq):
        nchunk = lax.div(nreq + _L - 1, jnp.int32(_L))

        def gbody(g, glenv):
            gbase = g * _GCAP
            def body(t, ng):
                base = t * _L
                iv = reqi_v[pl.ds(base, _L)]
                bv = reqb_v[pl.ds(base, _L)]
                m = (lax.shift_right_logical(iv, 16) == g) \
                    & ((base + iota) < nreq)
                plsc.store_compressed(gi_v.at[pl.ds(gbase + ng, _L)],
                                      iv, mask=m)
                plsc.store_compressed(gb_v.at[pl.ds(gbase + ng, _L)],
                                      bv, mask=m)
                cnt = plsc.all_reduce_population_count(m)
                return ng + cnt[0]
            ng = lax.fori_loop(0, nchunk, body, jnp.int32(0), unroll=False)
            return jnp.where(iota == g, ng, glenv)

        return lax.fori_loop(0, _NG, gbody, jnp.zeros((_L,), jnp.int32),
                             unroll=False)

    def run_table_pass(tabT_hbm, glenv, item_pass):
        def fire(blk_v, semb, j):
            it = wid + _NW * j
            @pl.when(it < _NB)
            def _():
                pltpu.make_async_copy(
                    tabT_hbm.at[:, pl.ds(it * _BW, _BW)], blk_v, semb).start()

        def waitblk(blk_v, semb, j):
            it = wid + _NW * j
            @pl.when(it < _NB)
            def _():
                pltpu.make_async_copy(
                    tabT_hbm.at[:, pl.ds(0, _BW)], blk_v, semb).wait()

        def emit_row(blk_v, half, r):
            iv = hiti_v[pl.ds(r, _L)]
            bv = hitb_v[pl.ds(r, _L)]
            il = iv[0] & (_BW - 1)
            bb = bv[0]
            ilv = il + jnp.zeros((_L,), jnp.int32)
            slot = half * (_SSLOT * EMBED_DIM) + r * EMBED_DIM
            for c in range(_NV):
                dv = lax.iota(jnp.int32, _L) + c * _L
                vals = plsc.load_gather(blk_v, [dv, ilv])
                stage_v[pl.ds(slot + c * _L, _L)] = vals
            src = stage_v.at[pl.ds(slot, EMBED_DIM)]
            if item_pass:
                @pl.when(bb < BATCH)
                def _():
                    pltpu.make_async_copy(
                        src, pemb.at[pl.ds(bb * EMBED_DIM, EMBED_DIM)],
                        semr).start()
                @pl.when(bb >= BATCH)
                def _():
                    pltpu.make_async_copy(
                        src,
                        nemb.at[pl.ds((bb - BATCH) * EMBED_DIM, EMBED_DIM)],
                        semr).start()
            else:
                pltpu.make_async_copy(
                    src, uemb.at[pl.ds(bb * EMBED_DIM, EMBED_DIM)],
                    semr).start()

        def gather_hits(g, glen, it, bval):
            gbase = g * _GCAP
            nchunk = lax.div(glen + _L - 1, jnp.int32(_L))
            def body(t, nh):
                base = t * _L
                iv = gi_v[pl.ds(gbase + base, _L)]
                bv = gb_v[pl.ds(gbase + base, _L)]
                itv = lax.shift_right_logical(iv, 7)
                m = (itv == it) & ((base + iota) < glen) & bval
                plsc.store_compressed(hiti_v.at[pl.ds(nh, _L)], iv, mask=m)
                plsc.store_compressed(hitb_v.at[pl.ds(nh, _L)], bv, mask=m)
                cnt = plsc.all_reduce_population_count(m)
                return nh + cnt[0]
            return lax.fori_loop(0, nchunk, body, jnp.int32(0),
                                 unroll=False)

        def drain_rows(nh):
            def body(r, _):
                pltpu.make_async_copy(
                    stage_v.at[pl.ds(0, EMBED_DIM)],
                    uemb.at[pl.ds(0, EMBED_DIM)], semr).wait()
                return 0
            lax.fori_loop(0, nh, body, 0, unroll=False)

        def process(blk_v, half, g, glen, j):
            it = wid + _NW * j
            nh = gather_hits(g, glen, it, it < _NB)
            def body(r, _):
                emit_row(blk_v, half, r)
                return 0
            lax.fori_loop(0, nh, body, 0, unroll=False)
            return nh

        fire(blk0_v, semb0, 0)
        fire(blk1_v, semb1, 1)

        def pair_body(jj, carry):
            ne, no = carry
            j0 = 2 * jj
            g = lax.shift_right_logical(j0, 4)
            glen = jnp.sum(jnp.where(iota == g, glenv, 0))
            waitblk(blk0_v, semb0, j0)
            drain_rows(ne)
            ne = process(blk0_v, 0, g, glen, j0)
            fire(blk0_v, semb0, j0 + 2)
            waitblk(blk1_v, semb1, j0 + 1)
            drain_rows(no)
            no = process(blk1_v, 1, g, glen, j0 + 1)
            fire(blk1_v, semb1, j0 + 3)
            return ne, no

        ne, no = lax.fori_loop(0, 123, pair_body,
                               (jnp.int32(0), jnp.int32(0)), unroll=False)
        drain_rows(ne)
        drain_rows(no)

        @pl.when(wid == (_NB % _NW))
        def _():
            pltpu.sync_copy(
                tabT_hbm.at[:, pl.ds(_NB * _BW, _RAG)], ragblk_v)
            glen15 = jnp.sum(jnp.where(iota == _NG - 1, glenv, 0))
            nh = gather_hits(_NG - 1, glen15, jnp.int32(_NB),
                             jnp.bool_(True))
            def body(r, _):
                emit_row(ragblk_v, 0, r)
                return 0
            lax.fori_loop(0, nh, body, 0, unroll=False)
            drain_rows(nh)

    nu = _scan_pass(users_hbm, idx_v, reqi_v, reqb_v, wid, 0, jnp.int32(0))
    run_table_pass(utabT_hbm, partition(nu), item_pass=False)

    np_ = _scan_pass(pos_hbm, idx_v, reqi_v, reqb_v, wid, 0, jnp.int32(0))
    ni = _scan_pass(neg_hbm, idx_v, reqi_v, reqb_v, wid, BATCH, np_)
    run_table_pass(itabT_hbm, partition(ni), item_pass=True)


def _compute_body(uemb_hbm, pemb_hbm, nemb_hbm,
                  pos_out, neg_out, reg_out,
                  urows_v, prows_v, nrows_v,
                  csp_v, csn_v, psc_v, nsc_v, acc_v):
    wid = lax.axis_index("s") * _NC + lax.axis_index("c")
    base = wid * _BPW
    fbase = base * EMBED_DIM
    pltpu.sync_copy(uemb_hbm.at[pl.ds(fbase, _BPW * EMBED_DIM)], urows_v)
    pltpu.sync_copy(pemb_hbm.at[pl.ds(fbase, _BPW * EMBED_DIM)], prows_v)
    pltpu.sync_copy(nemb_hbm.at[pl.ds(fbase, _BPW * EMBED_DIM)], nrows_v)

    lane15 = lax.iota(jnp.int32, _L) * _L + (_L - 1)

    def group_body(g, acc):
        for i in range(_GROUP):
            b = g * _GROUP + i
            dot_p = None
            dot_n = None
            for c in range(_NV):
                sl = pl.ds(b * EMBED_DIM + c * _L, _L)
                u = urows_v[sl]
                p = prows_v[sl]
                n = nrows_v[sl]
                acc = acc + u * u + p * p + n * n
                if dot_p is None:
                    dot_p = u * p
                    dot_n = u * n
                else:
                    dot_p = dot_p + u * p
                    dot_n = dot_n + u * n
            csp_v[pl.ds(i * _L, _L)] = plsc.cumsum(dot_p)
            csn_v[pl.ds(i * _L, _L)] = plsc.cumsum(dot_n)
        psc_v[pl.ds(g * _GROUP, _GROUP)] = plsc.load_gather(csp_v, [lane15])
        nsc_v[pl.ds(g * _GROUP, _GROUP)] = plsc.load_gather(csn_v, [lane15])
        return acc

    acc = lax.fori_loop(0, _BPW // _GROUP, group_body,
                        jnp.zeros((_L,), jnp.float32), unroll=False)
    acc_v[...] = acc

    pltpu.sync_copy(psc_v, pos_out.at[pl.ds(base, _BPW)])
    pltpu.sync_copy(nsc_v, neg_out.at[pl.ds(base, _BPW)])
    pltpu.sync_copy(acc_v, reg_out.at[wid])


@jax.jit
def _bprmf_sc(users, pos_items, neg_items, user_table, item_table):
    utabT = user_table.T
    itabT = item_table.T
    mesh = plsc.VectorSubcoreMesh(core_axis_name="c", subcore_axis_name="s")
    params = pltpu.CompilerParams(
        needs_layout_passes=False, use_tc_tiling_on_sc=True)

    extract = functools.partial(
        pl.kernel,
        mesh=mesh,
        compiler_params=params,
        out_type=(
            jax.ShapeDtypeStruct((BATCH * EMBED_DIM,), jnp.float32),
            jax.ShapeDtypeStruct((BATCH * EMBED_DIM,), jnp.float32),
            jax.ShapeDtypeStruct((BATCH * EMBED_DIM,), jnp.float32),
        ),
        scratch_types=[
            pltpu.VMEM((BATCH,), jnp.int32),
            pltpu.VMEM((_ICAP + _L,), jnp.int32),
            pltpu.VMEM((_ICAP + _L,), jnp.int32),
            pltpu.VMEM((_NG * _GCAP + _L,), jnp.int32),
            pltpu.VMEM((_NG * _GCAP + _L,), jnp.int32),
            pltpu.VMEM((64 + _L,), jnp.int32),
            pltpu.VMEM((64 + _L,), jnp.int32),
            pltpu.VMEM((EMBED_DIM, _BW), jnp.float32),
            pltpu.VMEM((EMBED_DIM, _BW), jnp.float32),
            pltpu.VMEM((EMBED_DIM, _RAG), jnp.float32),
            pltpu.VMEM((64 * EMBED_DIM,), jnp.float32),
            pltpu.SemaphoreType.DMA,
            pltpu.SemaphoreType.DMA,
            pltpu.SemaphoreType.DMA,
        ],
    )(_extract_body)
    uemb, pemb, nemb = extract(users, pos_items, neg_items, utabT, itabT)

    compute = functools.partial(
        pl.kernel,
        mesh=mesh,
        compiler_params=params,
        out_type=(
            jax.ShapeDtypeStruct((BATCH,), jnp.float32),
            jax.ShapeDtypeStruct((BATCH,), jnp.float32),
            jax.ShapeDtypeStruct((_NW, _L), jnp.float32),
        ),
        scratch_types=[
            pltpu.VMEM((_BPW * EMBED_DIM,), jnp.float32),
            pltpu.VMEM((_BPW * EMBED_DIM,), jnp.float32),
            pltpu.VMEM((_BPW * EMBED_DIM,), jnp.float32),
            pltpu.VMEM((_GROUP * _L,), jnp.float32),
            pltpu.VMEM((_GROUP * _L,), jnp.float32),
            pltpu.VMEM((_BPW,), jnp.float32),
            pltpu.VMEM((_BPW,), jnp.float32),
            pltpu.VMEM((_L,), jnp.float32),
        ],
    )(_compute_body)
    return compute(uemb, pemb, nemb)


def kernel(users, pos_items, neg_items, user_table, item_table):
    pos_scores, neg_scores, reg_part = _bprmf_sc(
        users, pos_items, neg_items, user_table, item_table)
    reg_loss = 0.5 * jnp.sum(reg_part) / float(BATCH)
    return (pos_scores, neg_scores, reg_loss)


# speedup vs baseline: 3.1546x; 1.2137x over previous
import functools

import jax
import jax.numpy as jnp
from jax import lax
from jax.experimental import pallas as pl
from jax.experimental.pallas import tpu as pltpu
from jax.experimental.pallas import tpu_sc as plsc

NUM_ROWS = 1000000
EMBED_DIM = 64
BATCH = 16384

_NC = 2
_NS = 16
_NW = _NC * _NS
_L = 16
_NV = EMBED_DIM // _L
_BW = 256
_BWLOG = 8
_NB = NUM_ROWS // _BW
_RAG = NUM_ROWS - _NB * _BW
_ICAP = 2048
_BPW = BATCH // _NW
_GROUP = 16


def _scan_pass(idx_hbm, idx_v, reqi_v, reqb_v, wid, b_off, off0):
    pltpu.sync_copy(idx_hbm, idx_v)
    iota = lax.iota(jnp.int32, _L)

    def body(k, off):
        iv = idx_v[pl.ds(k * _L, _L)]
        blkv = lax.shift_right_logical(iv, _BWLOG)
        m = (blkv & (_NW - 1)) == wid
        bv = k * _L + iota + b_off
        plsc.store_compressed(reqi_v.at[pl.ds(off, _L)], iv, mask=m)
        plsc.store_compressed(reqb_v.at[pl.ds(off, _L)], bv, mask=m)
        cnt = plsc.all_reduce_population_count(m)
        return off + cnt[0]

    return lax.fori_loop(0, BATCH // _L, body, off0, unroll=False)


_GCAP = 192
_NG = 16
_SSLOT = 48


def _extract_body(users_hbm, pos_hbm, neg_hbm, utabT_hbm, itabT_hbm,
                  uemb, pemb, nemb,
                  idx_v, reqi_v, reqb_v, gi_v, gb_v, hiti_v, hitb_v,
                  blk0_v, blk1_v, ragblk_v, stage_v, semb0, semb1, semr):
    wid = lax.axis_index("s") * _NC + lax.axis_index("c")
    iota = lax.iota(jnp.int32, _L)

    def partition(nreq):
        nchunk = lax.div(nreq + _L - 1, jnp.int32(_L))

        def gbody(g, glenv):
            gbase = g * _GCAP
            def body(t, ng):
                base = t * _L
                iv = reqi_v[pl.ds(base, _L)]
                bv = reqb_v[pl.ds(base, _L)]
                m = (lax.shift_right_logical(iv, 16) == g) \
                    & ((base + iota) < nreq)
                plsc.store_compressed(gi_v.at[pl.ds(gbase + ng, _L)],
                                      iv, mask=m)
                plsc.store_compressed(gb_v.at[pl.ds(gbase + ng, _L)],
                                      bv, mask=m)
                cnt = plsc.all_reduce_population_count(m)
                return ng + cnt[0]
            ng = lax.fori_loop(0, nchunk, body, jnp.int32(0), unroll=False)
            return jnp.where(iota == g, ng, glenv)

        return lax.fori_loop(0, _NG, gbody, jnp.zeros((_L,), jnp.int32),
                             unroll=False)

    def run_table_pass(tabT_hbm, glenv, item_pass):
        def fire(blk_v, semb, j):
            it = wid + _NW * j
            @pl.when(it < _NB)
            def _():
                pltpu.make_async_copy(
                    tabT_hbm.at[:, pl.ds(it * _BW, _BW)], blk_v, semb).start()

        def waitblk(blk_v, semb, j):
            it = wid + _NW * j
            @pl.when(it < _NB)
            def _():
                pltpu.make_async_copy(
                    tabT_hbm.at[:, pl.ds(0, _BW)], blk_v, semb).wait()

        def emit_row(blk_v, half, r):
            iv = hiti_v[pl.ds(r, _L)]
            bv = hitb_v[pl.ds(r, _L)]
            il = iv[0] & (_BW - 1)
            bb = bv[0]
            ilv = il + jnp.zeros((_L,), jnp.int32)
            slot = half * (_SSLOT * EMBED_DIM) + r * EMBED_DIM
            for c in range(_NV):
                dv = lax.iota(jnp.int32, _L) + c * _L
                vals = plsc.load_gather(blk_v, [dv, ilv])
                stage_v[pl.ds(slot + c * _L, _L)] = vals
            src = stage_v.at[pl.ds(slot, EMBED_DIM)]
            if item_pass:
                @pl.when(bb < BATCH)
                def _():
                    pltpu.make_async_copy(
                        src, pemb.at[pl.ds(bb * EMBED_DIM, EMBED_DIM)],
                        semr).start()
                @pl.when(bb >= BATCH)
                def _():
                    pltpu.make_async_copy(
                        src,
                        nemb.at[pl.ds((bb - BATCH) * EMBED_DIM, EMBED_DIM)],
                        semr).start()
            else:
                pltpu.make_async_copy(
                    src, uemb.at[pl.ds(bb * EMBED_DIM, EMBED_DIM)],
                    semr).start()

        def gather_hits(g, glen, it, bval):
            gbase = g * _GCAP
            nchunk = lax.div(glen + _L - 1, jnp.int32(_L))
            def body(t, nh):
                base = t * _L
                iv = gi_v[pl.ds(gbase + base, _L)]
                bv = gb_v[pl.ds(gbase + base, _L)]
                itv = lax.shift_right_logical(iv, _BWLOG)
                m = (itv == it) & ((base + iota) < glen) & bval
                plsc.store_compressed(hiti_v.at[pl.ds(nh, _L)], iv, mask=m)
                plsc.store_compressed(hitb_v.at[pl.ds(nh, _L)], bv, mask=m)
                cnt = plsc.all_reduce_population_count(m)
                return nh + cnt[0]
            return lax.fori_loop(0, nchunk, body, jnp.int32(0),
                                 unroll=False)

        def drain_rows(nh):
            def body(r, _):
                pltpu.make_async_copy(
                    stage_v.at[pl.ds(0, EMBED_DIM)],
                    uemb.at[pl.ds(0, EMBED_DIM)], semr).wait()
                return 0
            lax.fori_loop(0, nh, body, 0, unroll=False)

        def process(blk_v, half, g, glen, j):
            it = wid + _NW * j
            nh = gather_hits(g, glen, it, it < _NB)
            def body(r, _):
                emit_row(blk_v, half, r)
                return 0
            lax.fori_loop(0, nh, body, 0, unroll=False)
            return nh

        fire(blk0_v, semb0, 0)
        fire(blk1_v, semb1, 1)

        def pair_body(jj, carry):
            ne, no = carry
            j0 = 2 * jj
            g = lax.shift_right_logical(j0, 3)
            glen = jnp.sum(jnp.where(iota == g, glenv, 0))
            waitblk(blk0_v, semb0, j0)
            drain_rows(ne)
            ne = process(blk0_v, 0, g, glen, j0)
            fire(blk0_v, semb0, j0 + 2)
            waitblk(blk1_v, semb1, j0 + 1)
            drain_rows(no)
            no = process(blk1_v, 1, g, glen, j0 + 1)
            fire(blk1_v, semb1, j0 + 3)
            return ne, no

        ne, no = lax.fori_loop(0, 62, pair_body,
                               (jnp.int32(0), jnp.int32(0)), unroll=False)
        drain_rows(ne)
        drain_rows(no)

        @pl.when(wid == (_NB % _NW))
        def _():
            pltpu.sync_copy(
                tabT_hbm.at[:, pl.ds(_NB * _BW, _RAG)], ragblk_v)
            glen15 = jnp.sum(jnp.where(iota == _NG - 1, glenv, 0))
            nh = gather_hits(_NG - 1, glen15, jnp.int32(_NB),
                             jnp.bool_(True))
            def body(r, _):
                emit_row(ragblk_v, 0, r)
                return 0
            lax.fori_loop(0, nh, body, 0, unroll=False)
            drain_rows(nh)

    nu = _scan_pass(users_hbm, idx_v, reqi_v, reqb_v, wid, 0, jnp.int32(0))
    run_table_pass(utabT_hbm, partition(nu), item_pass=False)

    np_ = _scan_pass(pos_hbm, idx_v, reqi_v, reqb_v, wid, 0, jnp.int32(0))
    ni = _scan_pass(neg_hbm, idx_v, reqi_v, reqb_v, wid, BATCH, np_)
    run_table_pass(itabT_hbm, partition(ni), item_pass=True)


def _compute_body(uemb_hbm, pemb_hbm, nemb_hbm,
                  pos_out, neg_out, reg_out,
                  urows_v, prows_v, nrows_v,
                  csp_v, csn_v, psc_v, nsc_v, acc_v):
    wid = lax.axis_index("s") * _NC + lax.axis_index("c")
    base = wid * _BPW
    fbase = base * EMBED_DIM
    pltpu.sync_copy(uemb_hbm.at[pl.ds(fbase, _BPW * EMBED_DIM)], urows_v)
    pltpu.sync_copy(pemb_hbm.at[pl.ds(fbase, _BPW * EMBED_DIM)], prows_v)
    pltpu.sync_copy(nemb_hbm.at[pl.ds(fbase, _BPW * EMBED_DIM)], nrows_v)

    lane15 = lax.iota(jnp.int32, _L) * _L + (_L - 1)

    def group_body(g, acc):
        for i in range(_GROUP):
            b = g * _GROUP + i
            dot_p = None
            dot_n = None
            for c in range(_NV):
                sl = pl.ds(b * EMBED_DIM + c * _L, _L)
                u = urows_v[sl]
                p = prows_v[sl]
                n = nrows_v[sl]
                acc = acc + u * u + p * p + n * n
                if dot_p is None:
                    dot_p = u * p
                    dot_n = u * n
                else:
                    dot_p = dot_p + u * p
                    dot_n = dot_n + u * n
            csp_v[pl.ds(i * _L, _L)] = plsc.cumsum(dot_p)
            csn_v[pl.ds(i * _L, _L)] = plsc.cumsum(dot_n)
        psc_v[pl.ds(g * _GROUP, _GROUP)] = plsc.load_gather(csp_v, [lane15])
        nsc_v[pl.ds(g * _GROUP, _GROUP)] = plsc.load_gather(csn_v, [lane15])
        return acc

    acc = lax.fori_loop(0, _BPW // _GROUP, group_body,
                        jnp.zeros((_L,), jnp.float32), unroll=False)
    acc_v[...] = acc

    pltpu.sync_copy(psc_v, pos_out.at[pl.ds(base, _BPW)])
    pltpu.sync_copy(nsc_v, neg_out.at[pl.ds(base, _BPW)])
    pltpu.sync_copy(acc_v, reg_out.at[wid])


@jax.jit
def _bprmf_sc(users, pos_items, neg_items, user_table, item_table):
    utabT = user_table.T
    itabT = item_table.T
    mesh = plsc.VectorSubcoreMesh(core_axis_name="c", subcore_axis_name="s")
    params = pltpu.CompilerParams(
        needs_layout_passes=False, use_tc_tiling_on_sc=True)

    extract = functools.partial(
        pl.kernel,
        mesh=mesh,
        compiler_params=params,
        out_type=(
            jax.ShapeDtypeStruct((BATCH * EMBED_DIM,), jnp.float32),
            jax.ShapeDtypeStruct((BATCH * EMBED_DIM,), jnp.float32),
            jax.ShapeDtypeStruct((BATCH * EMBED_DIM,), jnp.float32),
        ),
        scratch_types=[
            pltpu.VMEM((BATCH,), jnp.int32),
            pltpu.VMEM((_ICAP + _L,), jnp.int32),
            pltpu.VMEM((_ICAP + _L,), jnp.int32),
            pltpu.VMEM((_NG * _GCAP + _L,), jnp.int32),
            pltpu.VMEM((_NG * _GCAP + _L,), jnp.int32),
            pltpu.VMEM((64 + _L,), jnp.int32),
            pltpu.VMEM((64 + _L,), jnp.int32),
            pltpu.VMEM((EMBED_DIM, _BW), jnp.float32),
            pltpu.VMEM((EMBED_DIM, _BW), jnp.float32),
            pltpu.VMEM((EMBED_DIM, _RAG), jnp.float32),
            pltpu.VMEM((2 * 48 * EMBED_DIM,), jnp.float32),
            pltpu.SemaphoreType.DMA,
            pltpu.SemaphoreType.DMA,
            pltpu.SemaphoreType.DMA,
        ],
    )(_extract_body)
    uemb, pemb, nemb = extract(users, pos_items, neg_items, utabT, itabT)

    compute = functools.partial(
        pl.kernel,
        mesh=mesh,
        compiler_params=params,
        out_type=(
            jax.ShapeDtypeStruct((BATCH,), jnp.float32),
            jax.ShapeDtypeStruct((BATCH,), jnp.float32),
            jax.ShapeDtypeStruct((_NW, _L), jnp.float32),
        ),
        scratch_types=[
            pltpu.VMEM((_BPW * EMBED_DIM,), jnp.float32),
            pltpu.VMEM((_BPW * EMBED_DIM,), jnp.float32),
            pltpu.VMEM((_BPW * EMBED_DIM,), jnp.float32),
            pltpu.VMEM((_GROUP * _L,), jnp.float32),
            pltpu.VMEM((_GROUP * _L,), jnp.float32),
            pltpu.VMEM((_BPW,), jnp.float32),
            pltpu.VMEM((_BPW,), jnp.float32),
            pltpu.VMEM((_L,), jnp.float32),
        ],
    )(_compute_body)
    return compute(uemb, pemb, nemb)


def kernel(users, pos_items, neg_items, user_table, item_table):
    pos_scores, neg_scores, reg_part = _bprmf_sc(
        users, pos_items, neg_items, user_table, item_table)
    reg_loss = 0.5 * jnp.sum(reg_part) / float(BATCH)
    return (pos_scores, neg_scores, reg_loss)


# speedup vs baseline: 3.4951x; 1.1079x over previous
import functools

import jax
import jax.numpy as jnp
from jax import lax
from jax.experimental import pallas as pl
from jax.experimental.pallas import tpu as pltpu
from jax.experimental.pallas import tpu_sc as plsc

NUM_ROWS = 1000000
EMBED_DIM = 64
BATCH = 16384

_NC = 2
_NS = 16
_NW = _NC * _NS
_L = 16
_NV = EMBED_DIM // _L
_BW = 512
_BWLOG = 9
_NB = NUM_ROWS // _BW
_RAG = NUM_ROWS - _NB * _BW
_ICAP = 2048
_BPW = BATCH // _NW
_GROUP = 16


def _scan_pass(idx_hbm, idx_v, reqi_v, reqb_v, wid, b_off, off0):
    pltpu.sync_copy(idx_hbm, idx_v)
    iota = lax.iota(jnp.int32, _L)

    def body(k, off):
        iv = idx_v[pl.ds(k * _L, _L)]
        blkv = lax.shift_right_logical(iv, _BWLOG)
        m = (blkv & (_NW - 1)) == wid
        bv = k * _L + iota + b_off
        plsc.store_compressed(reqi_v.at[pl.ds(off, _L)], iv, mask=m)
        plsc.store_compressed(reqb_v.at[pl.ds(off, _L)], bv, mask=m)
        cnt = plsc.all_reduce_population_count(m)
        return off + cnt[0]

    return lax.fori_loop(0, BATCH // _L, body, off0, unroll=4)


_GCAP = 192
_NG = 16
_SSLOT = 48


def _extract_body(users_hbm, pos_hbm, neg_hbm, utabT_hbm, itabT_hbm,
                  uemb, pemb, nemb,
                  idx_v, reqi_v, reqb_v, gi_v, gb_v, hiti_v, hitb_v,
                  blk0_v, blk1_v, ragblk_v, stage_v, semb0, semb1, semr):
    wid = lax.axis_index("s") * _NC + lax.axis_index("c")
    iota = lax.iota(jnp.int32, _L)

    def partition(nreq):
        nchunk = lax.div(nreq + _L - 1, jnp.int32(_L))

        def gbody(g, glenv):
            gbase = g * _GCAP
            def body(t, ng):
                base = t * _L
                iv = reqi_v[pl.ds(base, _L)]
                bv = reqb_v[pl.ds(base, _L)]
                m = (lax.shift_right_logical(iv, 16) == g) \
                    & ((base + iota) < nreq)
                plsc.store_compressed(gi_v.at[pl.ds(gbase + ng, _L)],
                                      iv, mask=m)
                plsc.store_compressed(gb_v.at[pl.ds(gbase + ng, _L)],
                                      bv, mask=m)
                cnt = plsc.all_reduce_population_count(m)
                return ng + cnt[0]
            ng = lax.fori_loop(0, nchunk, body, jnp.int32(0), unroll=False)
            return jnp.where(iota == g, ng, glenv)

        return lax.fori_loop(0, _NG, gbody, jnp.zeros((_L,), jnp.int32),
                             unroll=False)

    def run_table_pass(tabT_hbm, glenv, item_pass):
        def fire(blk_v, semb, j):
            it = wid + _NW * j
            @pl.when(it < _NB)
            def _():
                pltpu.make_async_copy(
                    tabT_hbm.at[:, pl.ds(it * _BW, _BW)], blk_v, semb).start()

        def waitblk(blk_v, semb, j):
            it = wid + _NW * j
            @pl.when(it < _NB)
            def _():
                pltpu.make_async_copy(
                    tabT_hbm.at[:, pl.ds(0, _BW)], blk_v, semb).wait()

        def emit_row(blk_v, half, r):
            iv = hiti_v[pl.ds(r, _L)]
            bv = hitb_v[pl.ds(r, _L)]
            il = iv[0] & (_BW - 1)
            bb = bv[0]
            ilv = il + jnp.zeros((_L,), jnp.int32)
            slot = half * (_SSLOT * EMBED_DIM) + r * EMBED_DIM
            for c in range(_NV):
                dv = lax.iota(jnp.int32, _L) + c * _L
                vals = plsc.load_gather(blk_v, [dv, ilv])
                stage_v[pl.ds(slot + c * _L, _L)] = vals
            src = stage_v.at[pl.ds(slot, EMBED_DIM)]
            if item_pass:
                @pl.when(bb < BATCH)
                def _():
                    pltpu.make_async_copy(
                        src, pemb.at[pl.ds(bb * EMBED_DIM, EMBED_DIM)],
                        semr).start()
                @pl.when(bb >= BATCH)
                def _():
                    pltpu.make_async_copy(
                        src,
                        nemb.at[pl.ds((bb - BATCH) * EMBED_DIM, EMBED_DIM)],
                        semr).start()
            else:
                pltpu.make_async_copy(
                    src, uemb.at[pl.ds(bb * EMBED_DIM, EMBED_DIM)],
                    semr).start()

        def gather_hits(g, glen, it, bval):
            gbase = g * _GCAP
            nchunk = lax.div(glen + _L - 1, jnp.int32(_L))
            def body(t, nh):
                base = t * _L
                iv = gi_v[pl.ds(gbase + base, _L)]
                bv = gb_v[pl.ds(gbase + base, _L)]
                itv = lax.shift_right_logical(iv, _BWLOG)
                m = (itv == it) & ((base + iota) < glen) & bval
                plsc.store_compressed(hiti_v.at[pl.ds(nh, _L)], iv, mask=m)
                plsc.store_compressed(hitb_v.at[pl.ds(nh, _L)], bv, mask=m)
                cnt = plsc.all_reduce_population_count(m)
                return nh + cnt[0]
            return lax.fori_loop(0, nchunk, body, jnp.int32(0),
                                 unroll=False)

        def drain_rows(nh):
            def body(r, _):
                pltpu.make_async_copy(
                    stage_v.at[pl.ds(0, EMBED_DIM)],
                    uemb.at[pl.ds(0, EMBED_DIM)], semr).wait()
                return 0
            lax.fori_loop(0, nh, body, 0, unroll=False)

        def process(blk_v, half, g, glen, j):
            it = wid + _NW * j
            nh = gather_hits(g, glen, it, it < _NB)
            def body(r, _):
                emit_row(blk_v, half, r)
                return 0
            lax.fori_loop(0, nh, body, 0, unroll=False)
            return nh

        fire(blk0_v, semb0, 0)
        fire(blk1_v, semb1, 1)

        def pair_body(jj, carry):
            ne, no = carry
            j0 = 2 * jj
            g = lax.shift_right_logical(j0, 2)
            glen = jnp.sum(jnp.where(iota == g, glenv, 0))
            waitblk(blk0_v, semb0, j0)
            drain_rows(ne)
            ne = process(blk0_v, 0, g, glen, j0)
            fire(blk0_v, semb0, j0 + 2)
            waitblk(blk1_v, semb1, j0 + 1)
            drain_rows(no)
            no = process(blk1_v, 1, g, glen, j0 + 1)
            fire(blk1_v, semb1, j0 + 3)
            return ne, no

        ne, no = lax.fori_loop(0, 31, pair_body,
                               (jnp.int32(0), jnp.int32(0)), unroll=False)
        drain_rows(ne)
        drain_rows(no)

        @pl.when(wid == (_NB % _NW))
        def _():
            pltpu.sync_copy(
                tabT_hbm.at[:, pl.ds(_NB * _BW, _RAG)], ragblk_v)
            glen15 = jnp.sum(jnp.where(iota == _NG - 1, glenv, 0))
            nh = gather_hits(_NG - 1, glen15, jnp.int32(_NB),
                             jnp.bool_(True))
            def body(r, _):
                emit_row(ragblk_v, 0, r)
                return 0
            lax.fori_loop(0, nh, body, 0, unroll=False)
            drain_rows(nh)

    nu = _scan_pass(users_hbm, idx_v, reqi_v, reqb_v, wid, 0, jnp.int32(0))
    run_table_pass(utabT_hbm, partition(nu), item_pass=False)

    np_ = _scan_pass(pos_hbm, idx_v, reqi_v, reqb_v, wid, 0, jnp.int32(0))
    ni = _scan_pass(neg_hbm, idx_v, reqi_v, reqb_v, wid, BATCH, np_)
    run_table_pass(itabT_hbm, partition(ni), item_pass=True)


def _compute_body(uemb_hbm, pemb_hbm, nemb_hbm,
                  pos_out, neg_out, reg_out,
                  urows_v, prows_v, nrows_v,
                  csp_v, csn_v, psc_v, nsc_v, acc_v):
    wid = lax.axis_index("s") * _NC + lax.axis_index("c")
    base = wid * _BPW
    fbase = base * EMBED_DIM
    pltpu.sync_copy(uemb_hbm.at[pl.ds(fbase, _BPW * EMBED_DIM)], urows_v)
    pltpu.sync_copy(pemb_hbm.at[pl.ds(fbase, _BPW * EMBED_DIM)], prows_v)
    pltpu.sync_copy(nemb_hbm.at[pl.ds(fbase, _BPW * EMBED_DIM)], nrows_v)

    lane15 = lax.iota(jnp.int32, _L) * _L + (_L - 1)

    def group_body(g, acc):
        for i in range(_GROUP):
            b = g * _GROUP + i
            dot_p = None
            dot_n = None
            for c in range(_NV):
                sl = pl.ds(b * EMBED_DIM + c * _L, _L)
                u = urows_v[sl]
                p = prows_v[sl]
                n = nrows_v[sl]
                acc = acc + u * u + p * p + n * n
                if dot_p is None:
                    dot_p = u * p
                    dot_n = u * n
                else:
                    dot_p = dot_p + u * p
                    dot_n = dot_n + u * n
            csp_v[pl.ds(i * _L, _L)] = plsc.cumsum(dot_p)
            csn_v[pl.ds(i * _L, _L)] = plsc.cumsum(dot_n)
        psc_v[pl.ds(g * _GROUP, _GROUP)] = plsc.load_gather(csp_v, [lane15])
        nsc_v[pl.ds(g * _GROUP, _GROUP)] = plsc.load_gather(csn_v, [lane15])
        return acc

    acc = lax.fori_loop(0, _BPW // _GROUP, group_body,
                        jnp.zeros((_L,), jnp.float32), unroll=False)
    acc_v[...] = acc

    pltpu.sync_copy(psc_v, pos_out.at[pl.ds(base, _BPW)])
    pltpu.sync_copy(nsc_v, neg_out.at[pl.ds(base, _BPW)])
    pltpu.sync_copy(acc_v, reg_out.at[wid])


@jax.jit
def _bprmf_sc(users, pos_items, neg_items, user_table, item_table):
    utabT = user_table.T
    itabT = item_table.T
    mesh = plsc.VectorSubcoreMesh(core_axis_name="c", subcore_axis_name="s")
    params = pltpu.CompilerParams(
        needs_layout_passes=False, use_tc_tiling_on_sc=True)

    extract = functools.partial(
        pl.kernel,
        mesh=mesh,
        compiler_params=params,
        out_type=(
            jax.ShapeDtypeStruct((BATCH * EMBED_DIM,), jnp.float32),
            jax.ShapeDtypeStruct((BATCH * EMBED_DIM,), jnp.float32),
            jax.ShapeDtypeStruct((BATCH * EMBED_DIM,), jnp.float32),
        ),
        scratch_types=[
            pltpu.VMEM((BATCH,), jnp.int32),
            pltpu.VMEM((_ICAP + _L,), jnp.int32),
            pltpu.VMEM((_ICAP + _L,), jnp.int32),
            pltpu.VMEM((_NG * _GCAP + _L,), jnp.int32),
            pltpu.VMEM((_NG * _GCAP + _L,), jnp.int32),
            pltpu.VMEM((64 + _L,), jnp.int32),
            pltpu.VMEM((64 + _L,), jnp.int32),
            pltpu.VMEM((EMBED_DIM, _BW), jnp.float32),
            pltpu.VMEM((EMBED_DIM, _BW), jnp.float32),
            pltpu.VMEM((EMBED_DIM, _RAG), jnp.float32),
            pltpu.VMEM((2 * 48 * EMBED_DIM,), jnp.float32),
            pltpu.SemaphoreType.DMA,
            pltpu.SemaphoreType.DMA,
            pltpu.SemaphoreType.DMA,
        ],
    )(_extract_body)
    uemb, pemb, nemb = extract(users, pos_items, neg_items, utabT, itabT)

    compute = functools.partial(
        pl.kernel,
        mesh=mesh,
        compiler_params=params,
        out_type=(
            jax.ShapeDtypeStruct((BATCH,), jnp.float32),
            jax.ShapeDtypeStruct((BATCH,), jnp.float32),
            jax.ShapeDtypeStruct((_NW, _L), jnp.float32),
        ),
        scratch_types=[
            pltpu.VMEM((_BPW * EMBED_DIM,), jnp.float32),
            pltpu.VMEM((_BPW * EMBED_DIM,), jnp.float32),
            pltpu.VMEM((_BPW * EMBED_DIM,), jnp.float32),
            pltpu.VMEM((_GROUP * _L,), jnp.float32),
            pltpu.VMEM((_GROUP * _L,), jnp.float32),
            pltpu.VMEM((_BPW,), jnp.float32),
            pltpu.VMEM((_BPW,), jnp.float32),
            pltpu.VMEM((_L,), jnp.float32),
        ],
    )(_compute_body)
    return compute(uemb, pemb, nemb)


def kernel(users, pos_items, neg_items, user_table, item_table):
    pos_scores, neg_scores, reg_part = _bprmf_sc(
        users, pos_items, neg_items, user_table, item_table)
    reg_loss = 0.5 * jnp.sum(reg_part) / float(BATCH)
    return (pos_scores, neg_scores, reg_loss)
